# baseline (speedup 1.0000x reference)
import functools

import jax
import jax.numpy as jnp
from jax import lax
from jax.experimental import pallas as pl
from jax.experimental.pallas import tpu as pltpu
from jax.experimental.pallas import tpu_sc as plsc

N = 10000
E = 160000
D = 128
XP = 16
NLAYERS = 4

CH = 128
NW = 32
CPW = 40
EP = NW * CPW * CH
NCHUNK = EP // CH
RPS = N // 16
ZR = 25

BF = jnp.bfloat16
F32 = jnp.float32


def _mesh():
    return plsc.VectorSubcoreMesh(core_axis_name="c", subcore_axis_name="s")


def _sc_gather(ph, pc, xt, row3, col3):
    @functools.partial(
        pl.kernel,
        out_type=(
            jax.ShapeDtypeStruct((EP, D), F32),
            jax.ShapeDtypeStruct((EP, D), F32),
            jax.ShapeDtypeStruct((EP, XP), F32),
            jax.ShapeDtypeStruct((EP, XP), F32),
        ),
        mesh=_mesh(),
        compiler_params=pltpu.CompilerParams(use_tc_tiling_on_sc=False),
        scratch_types=[
            pltpu.VMEM((1, CH), jnp.int32),
            pltpu.VMEM((1, CH), jnp.int32),
            pltpu.VMEM((CH, D), F32),
            pltpu.VMEM((CH, D), F32),
            pltpu.VMEM((CH, XP), F32),
            pltpu.VMEM((CH, XP), F32),
        ],
    )
    def gk(ph_h, pc_h, xt_h, row_h, col_h,
           phr_h, pcc_h, xr_h, xc_h,
           rowb, colb, bh1, bh2, bx1, bx2):
        c = lax.axis_index("c")
        s = lax.axis_index("s")
        w = s * 2 + c

        @pl.loop(0, CPW)
        def _(k):
            base = (k * NW + w) * CH
            pltpu.sync_copy(row_h.at[w, pl.ds(k, 1)], rowb)
            pltpu.sync_copy(col_h.at[w, pl.ds(k, 1)], colb)
            pltpu.sync_copy(ph_h.at[rowb.at[0]], bh1)
            pltpu.sync_copy(pc_h.at[colb.at[0]], bh2)
            pltpu.sync_copy(xt_h.at[rowb.at[0]], bx1)
            pltpu.sync_copy(xt_h.at[colb.at[0]], bx2)
            pltpu.sync_copy(bh1, phr_h.at[pl.ds(base, CH)])
            pltpu.sync_copy(bh2, pcc_h.at[pl.ds(base, CH)])
            pltpu.sync_copy(bx1, xr_h.at[pl.ds(base, CH)])
            pltpu.sync_copy(bx2, xc_h.at[pl.ds(base, CH)])


    return gk(ph, pc, xt, row3, col3)


def _sc_scatter(m, t16, row3):
    @functools.partial(
        pl.kernel,
        out_type=(
            jax.ShapeDtypeStruct((2, N, D), F32),
            jax.ShapeDtypeStruct((2, N, XP), F32),
        ),
        mesh=_mesh(),
        compiler_params=pltpu.CompilerParams(use_tc_tiling_on_sc=False),
        scratch_types=[
            pltpu.VMEM((1, CH), jnp.int32),
            pltpu.VMEM((CH, D), F32),
            pltpu.VMEM((CH, XP), F32),
            pltpu.VMEM((ZR, D), F32),
            pltpu.VMEM((ZR, XP), F32),
            pltpu.VMEM_SHARED((N, D), F32),
            pltpu.VMEM_SHARED((N, XP), F32),
        ],
    )
    def sk(m_h, t_h, row_h, agg_h, tagg_h,
           rowb, mb, tb, zd, zx, agg_sh, tagg_sh):
        c = lax.axis_index("c")
        s = lax.axis_index("s")
        w = s * 2 + c
        @pl.loop(0, ZR)
        def _(r):
            for g in range(D // 16):
                zd[r, pl.ds(g * 16, 16)] = jnp.zeros((16,), F32)
            zx[r, pl.ds(0, 16)] = jnp.zeros((16,), F32)

        @pl.loop(0, RPS // ZR)
        def _(kk):
            off = s * RPS + kk * ZR
            pltpu.sync_copy(zd, agg_sh.at[pl.ds(off, ZR)])
            pltpu.sync_copy(zx, tagg_sh.at[pl.ds(off, ZR)])

        plsc.subcore_barrier()

        @pl.loop(0, CPW)
        def _(k):
            base = (k * NW + w) * CH
            pltpu.sync_copy(row_h.at[w, pl.ds(k, 1)], rowb)
            pltpu.sync_copy(m_h.at[pl.ds(base, CH)], mb)
            pltpu.sync_copy(t_h.at[pl.ds(base, CH)], tb)
            pltpu.sync_copy(mb, agg_sh.at[rowb.at[0]], add=True)
            pltpu.sync_copy(tb, tagg_sh.at[rowb.at[0]], add=True)

        plsc.subcore_barrier()
        off = s * RPS
        pltpu.sync_copy(agg_sh.at[pl.ds(off, RPS)],
                        agg_h.at[c, pl.ds(off, RPS)])
        pltpu.sync_copy(tagg_sh.at[pl.ds(off, RPS)],
                        tagg_h.at[c, pl.ds(off, RPS)])

    return sk(m, t16, row3)


def _silu(v):
    return v * jax.nn.sigmoid(v)


_BE = 2048
_BN = 2000


def _edge_tc(phr, pcc, xr, xc, ea, w1c, W1dT, W2T, b2, Wc1T, bc1, wc2):
    def body(phr_r, pcc_r, xr_r, xc_r, ea_r,
             w1c_r, W1dT_r, W2T_r, b2_r, Wc1T_r, bc1_r, wc2_r,
             m_r, mb_r, t_r):
        d = xr_r[...] - xc_r[...]
        radial = jnp.sum(d * d, axis=1, keepdims=True)
        z1 = (phr_r[...] + pcc_r[...] + radial * w1c_r[...]
              + jnp.dot(ea_r[...], W1dT_r[...],
                        preferred_element_type=F32))
        a1 = _silu(z1)
        m = _silu(jnp.dot(a1.astype(BF), W2T_r[...],
                          preferred_element_type=F32) + b2_r[...])
        cc = _silu(jnp.dot(m.astype(BF), Wc1T_r[...],
                           preferred_element_type=F32) + bc1_r[...])
        sval = jnp.sum(cc * wc2_r[...], axis=1, keepdims=True)
        t = d * sval
        lane = lax.broadcasted_iota(jnp.int32, (_BE, XP), 1)
        t = jnp.where(lane == 3, 1.0, t)
        gidx = (pl.program_id(0) * _BE
                + lax.broadcasted_iota(jnp.int32, (_BE, 1), 0))
        valid = gidx < E
        m = jnp.where(valid, m, 0.0)
        t = jnp.where(valid, t, 0.0)
        m_r[...] = m
        mb_r[...] = m.astype(BF)
        t_r[...] = t

    wspec = pl.BlockSpec((D, D), lambda i: (0, 0))
    vspec = pl.BlockSpec((1, D), lambda i: (0, 0))
    return pl.pallas_call(
        body,
        grid=(EP // _BE,),
        in_specs=[
            pl.BlockSpec((_BE, D), lambda i: (i, 0)),
            pl.BlockSpec((_BE, D), lambda i: (i, 0)),
            pl.BlockSpec((_BE, XP), lambda i: (i, 0)),
            pl.BlockSpec((_BE, XP), lambda i: (i, 0)),
            pl.BlockSpec((_BE, D), lambda i: (i, 0)),
            vspec, wspec, wspec, vspec, wspec, vspec, vspec,
        ],
        out_specs=[
            pl.BlockSpec((_BE, D), lambda i: (i, 0)),
            pl.BlockSpec((_BE, D), lambda i: (i, 0)),
            pl.BlockSpec((_BE, XP), lambda i: (i, 0)),
        ],
        out_shape=[
            jax.ShapeDtypeStruct((EP, D), F32),
            jax.ShapeDtypeStruct((EP, D), BF),
            jax.ShapeDtypeStruct((EP, XP), F32),
        ],
    )(phr, pcc, xr, xc, ea, w1c, W1dT, W2T, b2, Wc1T, bc1, wc2)


def _node_tc(h, xp, pa, pt, n1hT, n1aT, b1n, n2T, b2n, WaT, ba, WbT):
    def body(h_r, xp_r, pa_r, pt_r,
             n1hT_r, n1aT_r, b1n_r, n2T_r, b2n_r, WaT_r, ba_r, WbT_r,
             hn_r, xn_r, ph_r, pc_r):
        agg = pa_r[0] + pa_r[1]
        ts = pt_r[0] + pt_r[1]
        cnt = jnp.maximum(ts[:, 3:4], 1.0)
        lane = lax.broadcasted_iota(jnp.int32, (_BN, XP), 1)
        xn = xp_r[...] + jnp.where(lane < 3, ts / cnt, 0.0)
        xn_r[...] = xn
        h = h_r[...]
        z = (jnp.dot(h.astype(BF), n1hT_r[...], preferred_element_type=F32)
             + jnp.dot(agg.astype(BF), n1aT_r[...],
                       preferred_element_type=F32)
             + b1n_r[...])
        hn = h + jnp.dot(_silu(z).astype(BF), n2T_r[...],
                         preferred_element_type=F32) + b2n_r[...]
        hn_r[...] = hn
        hb = hn.astype(BF)
        ph_r[...] = jnp.dot(hb, WaT_r[...],
                            preferred_element_type=F32) + ba_r[...]
        pc_r[...] = jnp.dot(hb, WbT_r[...],
                            preferred_element_type=F32)

    wspec = pl.BlockSpec((D, D), lambda i: (0, 0))
    vspec = pl.BlockSpec((1, D), lambda i: (0, 0))
    return pl.pallas_call(
        body,
        grid=(N // _BN,),
        in_specs=[
            pl.BlockSpec((_BN, D), lambda i: (i, 0)),
            pl.BlockSpec((_BN, XP), lambda i: (i, 0)),
            pl.BlockSpec((2, _BN, D), lambda i: (0, i, 0)),
            pl.BlockSpec((2, _BN, XP), lambda i: (0, i, 0)),
            wspec, wspec, vspec, wspec, vspec, wspec, vspec, wspec,
        ],
        out_specs=[
            pl.BlockSpec((_BN, D), lambda i: (i, 0)),
            pl.BlockSpec((_BN, XP), lambda i: (i, 0)),
            pl.BlockSpec((_BN, D), lambda i: (i, 0)),
            pl.BlockSpec((_BN, D), lambda i: (i, 0)),
        ],
        out_shape=[
            jax.ShapeDtypeStruct((N, D), F32),
            jax.ShapeDtypeStruct((N, XP), F32),
            jax.ShapeDtypeStruct((N, D), F32),
            jax.ShapeDtypeStruct((N, D), F32),
        ],
    )(h, xp, pa, pt, n1hT, n1aT, b1n, n2T, b2n, WaT, ba, WbT)


def _prologue_tc(h, WembT, bemb, WaT, ba, WbT):
    def body(h_r, WembT_r, bemb_r, WaT_r, ba_r, WbT_r, h0_r, ph_r, pc_r):
        h0 = jnp.dot(h_r[...].astype(BF), WembT_r[...],
                     preferred_element_type=F32) + bemb_r[...]
        h0_r[...] = h0
        hb = h0.astype(BF)
        ph_r[...] = jnp.dot(hb, WaT_r[...],
                            preferred_element_type=F32) + ba_r[...]
        pc_r[...] = jnp.dot(hb, WbT_r[...],
                            preferred_element_type=F32)

    wspec = pl.BlockSpec((D, D), lambda i: (0, 0))
    vspec = pl.BlockSpec((1, D), lambda i: (0, 0))
    bspec = pl.BlockSpec((_BN, D), lambda i: (i, 0))
    return pl.pallas_call(
        body,
        grid=(N // _BN,),
        in_specs=[bspec, wspec, vspec, wspec, vspec, wspec],
        out_specs=[bspec, bspec, bspec],
        out_shape=[
            jax.ShapeDtypeStruct((N, D), F32),
            jax.ShapeDtypeStruct((N, D), F32),
            jax.ShapeDtypeStruct((N, D), F32),
        ],
    )(h, WembT, bemb, WaT, ba, WbT)


def _prep_weights(lp):
    W1 = lp["edge1"]["W"]
    return dict(
        WaT=W1[:, :D].T.astype(BF),
        ba=lp["edge1"]["b"][None],
        WbT=W1[:, D:2 * D].T.astype(BF),
        w1c=W1[:, 2 * D][None],
        W1dT=W1[:, 2 * D + 1:].T.astype(BF),
        W2T=lp["edge2"]["W"].T.astype(BF),
        b2=lp["edge2"]["b"][None],
        Wc1T=lp["coord1"]["W"].T.astype(BF),
        bc1=lp["coord1"]["b"][None],
        wc2=lp["coord2"]["W"],
        n1hT=lp["node1"]["W"][:, :D].T.astype(BF),
        n1aT=lp["node1"]["W"][:, D:].T.astype(BF),
        b1n=lp["node1"]["b"][None],
        n2T=lp["node2"]["W"].T.astype(BF),
        b2n=lp["node2"]["b"][None],
    )


def kernel(h, x, edges, edge_attr, params):
    rp = jnp.pad(edges[0], (0, EP - E))
    cp = jnp.pad(edges[1], (0, EP - E))
    row3 = rp.reshape(CPW, NW, CH).transpose(1, 0, 2)
    col3 = cp.reshape(CPW, NW, CH).transpose(1, 0, 2)
    xt = jnp.pad(x, ((0, 0), (0, XP - 3)))
    ea = jnp.pad(edge_attr, ((0, EP - E), (0, 0))).astype(BF)
    ws = [_prep_weights(lp) for lp in params["layers"]]
    h0, ph, pc = _prologue_tc(
        h, params["emb"]["W"].T.astype(BF), params["emb"]["b"][None],
        ws[0]["WaT"], ws[0]["ba"], ws[0]["WbT"])
    hcur, xcur = h0, xt
    for l in range(NLAYERS):
        w = ws[l]
        phr, pcc, xr, xc = _sc_gather(ph, pc, xcur, row3, col3)
        m, m_bf, t16 = _edge_tc(phr, pcc, xr, xc, ea,
                                w["w1c"], w["W1dT"], w["W2T"], w["b2"],
                                w["Wc1T"], w["bc1"], w["wc2"])
        pa, pt = _sc_scatter(m, t16, row3)
        wn = ws[l + 1] if l + 1 < NLAYERS else ws[0]
        hcur, xcur, ph, pc = _node_tc(
            hcur, xcur, pa, pt,
            w["n1hT"], w["n1aT"], w["b1n"], w["n2T"], w["b2n"],
            wn["WaT"], wn["ba"], wn["WbT"])
        ea = m_bf
    return hcur


# speedup vs baseline: 1.3182x; 1.3182x over previous
import functools

import jax
import jax.numpy as jnp
from jax import lax
from jax.experimental import pallas as pl
from jax.experimental.pallas import tpu as pltpu
from jax.experimental.pallas import tpu_sc as plsc

N = 10000
E = 160000
D = 128
XP = 16
NLAYERS = 4

CH = 128
NCHUNK = E // CH
NW = 32
CPW = (NCHUNK + NW - 1) // NW
RPS = N // 16
ZR = 25


def _mesh():
    return plsc.VectorSubcoreMesh(core_axis_name="c", subcore_axis_name="s")


def _sc_gather(ph, pc, xp, row2, col2):
    @functools.partial(
        pl.kernel,
        out_type=(
            jax.ShapeDtypeStruct((E, D), jnp.float32),
            jax.ShapeDtypeStruct((E, D), jnp.float32),
            jax.ShapeDtypeStruct((E, XP), jnp.float32),
            jax.ShapeDtypeStruct((E, XP), jnp.float32),
        ),
        mesh=_mesh(),
        compiler_params=pltpu.CompilerParams(use_tc_tiling_on_sc=False),
        scratch_types=[
            pltpu.VMEM((1, CH), jnp.int32),
            pltpu.VMEM((1, CH), jnp.int32),
            pltpu.VMEM((CH, D), jnp.float32),
            pltpu.VMEM((CH, D), jnp.float32),
            pltpu.VMEM((CH, XP), jnp.float32),
            pltpu.VMEM((CH, XP), jnp.float32),
        ],
    )
    def gk(ph_h, pc_h, xp_h, row_h, col_h,
           phr_h, pcc_h, xr_h, xc_h,
           rowb, colb, bh1, bh2, bx1, bx2):
        c = lax.axis_index("c")
        s = lax.axis_index("s")
        wid = s * 2 + c

        @pl.loop(0, CPW)
        def _(k):
            j = wid + k * NW

            @pl.when(j < NCHUNK)
            def _():
                base = j * CH
                pltpu.sync_copy(row_h.at[pl.ds(j, 1)], rowb)
                pltpu.sync_copy(col_h.at[pl.ds(j, 1)], colb)
                pltpu.sync_copy(ph_h.at[rowb.at[0]], bh1)
                pltpu.sync_copy(pc_h.at[colb.at[0]], bh2)
                pltpu.sync_copy(xp_h.at[rowb.at[0]], bx1)
                pltpu.sync_copy(xp_h.at[colb.at[0]], bx2)
                pltpu.sync_copy(bh1, phr_h.at[pl.ds(base, CH)])
                pltpu.sync_copy(bh2, pcc_h.at[pl.ds(base, CH)])
                pltpu.sync_copy(bx1, xr_h.at[pl.ds(base, CH)])
                pltpu.sync_copy(bx2, xc_h.at[pl.ds(base, CH)])

    return gk(ph, pc, xp, row2, col2)


def _sc_scatter(m, t16, row2):
    @functools.partial(
        pl.kernel,
        out_type=(
            jax.ShapeDtypeStruct((2, N, D), jnp.float32),
            jax.ShapeDtypeStruct((2, N, XP), jnp.float32),
        ),
        mesh=_mesh(),
        compiler_params=pltpu.CompilerParams(use_tc_tiling_on_sc=False),
        scratch_types=[
            pltpu.VMEM((1, CH), jnp.int32),
            pltpu.VMEM((CH, D), jnp.float32),
            pltpu.VMEM((CH, XP), jnp.float32),
            pltpu.VMEM((ZR, D), jnp.float32),
            pltpu.VMEM((ZR, XP), jnp.float32),
            pltpu.VMEM_SHARED((N, D), jnp.float32),
            pltpu.VMEM_SHARED((N, XP), jnp.float32),
        ],
    )
    def sk(m_h, t_h, row_h, agg_h, tagg_h,
           rowb, mb, tb, zd, zx, agg_sh, tagg_sh):
        c = lax.axis_index("c")
        s = lax.axis_index("s")
        wid = s * 2 + c

        @pl.loop(0, ZR)
        def _(r):
            for g in range(D // 16):
                zd[r, pl.ds(g * 16, 16)] = jnp.zeros((16,), jnp.float32)
            zx[r, pl.ds(0, 16)] = jnp.zeros((16,), jnp.float32)

        @pl.loop(0, RPS // ZR)
        def _(kk):
            off = s * RPS + kk * ZR
            pltpu.sync_copy(zd, agg_sh.at[pl.ds(off, ZR)])
            pltpu.sync_copy(zx, tagg_sh.at[pl.ds(off, ZR)])

        plsc.subcore_barrier()

        @pl.loop(0, CPW)
        def _(k):
            j = wid + k * NW

            @pl.when(j < NCHUNK)
            def _():
                base = j * CH
                pltpu.sync_copy(row_h.at[pl.ds(j, 1)], rowb)
                pltpu.sync_copy(m_h.at[pl.ds(base, CH)], mb)
                pltpu.sync_copy(t_h.at[pl.ds(base, CH)], tb)
                pltpu.sync_copy(mb, agg_sh.at[rowb.at[0]], add=True)
                pltpu.sync_copy(tb, tagg_sh.at[rowb.at[0]], add=True)

        plsc.subcore_barrier()
        off = s * RPS
        pltpu.sync_copy(agg_sh.at[pl.ds(off, RPS)],
                        agg_h.at[c, pl.ds(off, RPS)])
        pltpu.sync_copy(tagg_sh.at[pl.ds(off, RPS)],
                        tagg_h.at[c, pl.ds(off, RPS)])

    return sk(m, t16, row2)


def _silu(v):
    return v * jax.nn.sigmoid(v)


_BE = 2000
_BN = 2000


def _edge_tc(phr, pcc, xr, xc, ea, w1c, W1dT, W2T, b2, Wc1T, bc1, wc2):
    def body(phr_r, pcc_r, xr_r, xc_r, ea_r,
             w1c_r, W1dT_r, W2T_r, b2_r, Wc1T_r, bc1_r, wc2_r,
             m_r, t_r):
        d = xr_r[...] - xc_r[...]
        radial = jnp.sum(d * d, axis=1, keepdims=True)
        z1 = (phr_r[...] + pcc_r[...] + radial * w1c_r[...]
              + jnp.dot(ea_r[...], W1dT_r[...],
                        preferred_element_type=jnp.float32))
        a1 = _silu(z1)
        m = _silu(jnp.dot(a1, W2T_r[...],
                          preferred_element_type=jnp.float32) + b2_r[...])
        cc = _silu(jnp.dot(m, Wc1T_r[...],
                           preferred_element_type=jnp.float32) + bc1_r[...])
        sval = jnp.sum(cc * wc2_r[...], axis=1, keepdims=True)
        t = d * sval
        lane = lax.broadcasted_iota(jnp.int32, (_BE, XP), 1)
        t = jnp.where(lane == 3, 1.0, t)
        m_r[...] = m
        t_r[...] = t

    wspec = pl.BlockSpec((D, D), lambda i: (0, 0))
    vspec = pl.BlockSpec((1, D), lambda i: (0, 0))
    return pl.pallas_call(
        body,
        grid=(E // _BE,),
        in_specs=[
            pl.BlockSpec((_BE, D), lambda i: (i, 0)),
            pl.BlockSpec((_BE, D), lambda i: (i, 0)),
            pl.BlockSpec((_BE, XP), lambda i: (i, 0)),
            pl.BlockSpec((_BE, XP), lambda i: (i, 0)),
            pl.BlockSpec((_BE, D), lambda i: (i, 0)),
            vspec, wspec, wspec, vspec, wspec, vspec, vspec,
        ],
        out_specs=[
            pl.BlockSpec((_BE, D), lambda i: (i, 0)),
            pl.BlockSpec((_BE, XP), lambda i: (i, 0)),
        ],
        out_shape=[
            jax.ShapeDtypeStruct((E, D), jnp.float32),
            jax.ShapeDtypeStruct((E, XP), jnp.float32),
        ],
    )(phr, pcc, xr, xc, ea, w1c, W1dT, W2T, b2, Wc1T, bc1, wc2)


def _node_tc(h, xp, pa, pt, n1hT, n1aT, b1n, n2T, b2n, WaT, ba, WbT):
    def body(h_r, xp_r, pa_r, pt_r,
             n1hT_r, n1aT_r, b1n_r, n2T_r, b2n_r, WaT_r, ba_r, WbT_r,
             hn_r, xn_r, ph_r, pc_r):
        agg = pa_r[0] + pa_r[1]
        ts = pt_r[0] + pt_r[1]
        cnt = jnp.maximum(ts[:, 3:4], 1.0)
        lane = lax.broadcasted_iota(jnp.int32, (_BN, XP), 1)
        xn_r[...] = xp_r[...] + jnp.where(lane < 3, ts / cnt, 0.0)
        h = h_r[...]
        z = (jnp.dot(h, n1hT_r[...], preferred_element_type=jnp.float32)
             + jnp.dot(agg, n1aT_r[...], preferred_element_type=jnp.float32)
             + b1n_r[...])
        hn = h + jnp.dot(_silu(z), n2T_r[...],
                         preferred_element_type=jnp.float32) + b2n_r[...]
        hn_r[...] = hn
        ph_r[...] = jnp.dot(hn, WaT_r[...],
                            preferred_element_type=jnp.float32) + ba_r[...]
        pc_r[...] = jnp.dot(hn, WbT_r[...],
                            preferred_element_type=jnp.float32)

    wspec = pl.BlockSpec((D, D), lambda i: (0, 0))
    vspec = pl.BlockSpec((1, D), lambda i: (0, 0))
    return pl.pallas_call(
        body,
        grid=(N // _BN,),
        in_specs=[
            pl.BlockSpec((_BN, D), lambda i: (i, 0)),
            pl.BlockSpec((_BN, XP), lambda i: (i, 0)),
            pl.BlockSpec((2, _BN, D), lambda i: (0, i, 0)),
            pl.BlockSpec((2, _BN, XP), lambda i: (0, i, 0)),
            wspec, wspec, vspec, wspec, vspec, wspec, vspec, wspec,
        ],
        out_specs=[
            pl.BlockSpec((_BN, D), lambda i: (i, 0)),
            pl.BlockSpec((_BN, XP), lambda i: (i, 0)),
            pl.BlockSpec((_BN, D), lambda i: (i, 0)),
            pl.BlockSpec((_BN, D), lambda i: (i, 0)),
        ],
        out_shape=[
            jax.ShapeDtypeStruct((N, D), jnp.float32),
            jax.ShapeDtypeStruct((N, XP), jnp.float32),
            jax.ShapeDtypeStruct((N, D), jnp.float32),
            jax.ShapeDtypeStruct((N, D), jnp.float32),
        ],
    )(h, xp, pa, pt, n1hT, n1aT, b1n, n2T, b2n, WaT, ba, WbT)


def _prologue_tc(h, WembT, bemb, WaT, ba, WbT):
    def body(h_r, WembT_r, bemb_r, WaT_r, ba_r, WbT_r, h0_r, ph_r, pc_r):
        h0 = jnp.dot(h_r[...], WembT_r[...],
                     preferred_element_type=jnp.float32) + bemb_r[...]
        h0_r[...] = h0
        ph_r[...] = jnp.dot(h0, WaT_r[...],
                            preferred_element_type=jnp.float32) + ba_r[...]
        pc_r[...] = jnp.dot(h0, WbT_r[...],
                            preferred_element_type=jnp.float32)

    wspec = pl.BlockSpec((D, D), lambda i: (0, 0))
    vspec = pl.BlockSpec((1, D), lambda i: (0, 0))
    bspec = pl.BlockSpec((_BN, D), lambda i: (i, 0))
    return pl.pallas_call(
        body,
        grid=(N // _BN,),
        in_specs=[bspec, wspec, vspec, wspec, vspec, wspec],
        out_specs=[bspec, bspec, bspec],
        out_shape=[
            jax.ShapeDtypeStruct((N, D), jnp.float32),
            jax.ShapeDtypeStruct((N, D), jnp.float32),
            jax.ShapeDtypeStruct((N, D), jnp.float32),
        ],
    )(h, WembT, bemb, WaT, ba, WbT)


def _prep_weights(lp):
    W1 = lp["edge1"]["W"]
    return dict(
        WaT=W1[:, :D].T,
        ba=lp["edge1"]["b"][None],
        WbT=W1[:, D:2 * D].T,
        w1c=W1[:, 2 * D][None],
        W1dT=W1[:, 2 * D + 1:].T,
        W2T=lp["edge2"]["W"].T,
        b2=lp["edge2"]["b"][None],
        Wc1T=lp["coord1"]["W"].T,
        bc1=lp["coord1"]["b"][None],
        wc2=lp["coord2"]["W"],
        n1hT=lp["node1"]["W"][:, :D].T,
        n1aT=lp["node1"]["W"][:, D:].T,
        b1n=lp["node1"]["b"][None],
        n2T=lp["node2"]["W"].T,
        b2n=lp["node2"]["b"][None],
    )


def kernel(h, x, edges, edge_attr, params):
    row2 = edges[0].reshape(NCHUNK, CH)
    col2 = edges[1].reshape(NCHUNK, CH)
    xp = jnp.pad(x, ((0, 0), (0, XP - 3)))
    ws = [_prep_weights(lp) for lp in params["layers"]]
    h0, ph, pc = _prologue_tc(
        h, params["emb"]["W"].T, params["emb"]["b"][None],
        ws[0]["WaT"], ws[0]["ba"], ws[0]["WbT"])
    hcur, xcur, ea = h0, xp, edge_attr
    for l in range(NLAYERS):
        w = ws[l]
        phr, pcc, xr, xc = _sc_gather(ph, pc, xcur, row2, col2)
        m, t16 = _edge_tc(phr, pcc, xr, xc, ea,
                          w["w1c"], w["W1dT"], w["W2T"], w["b2"],
                          w["Wc1T"], w["bc1"], w["wc2"])
        pa, pt = _sc_scatter(m, t16, row2)
        wn = ws[l + 1] if l + 1 < NLAYERS else ws[0]
        hcur, xcur, ph, pc = _node_tc(
            hcur, xcur, pa, pt,
            w["n1hT"], w["n1aT"], w["b1n"], w["n2T"], w["b2n"],
            wn["WaT"], wn["ba"], wn["WbT"])
        ea = m
    return hcur


# speedup vs baseline: 1.6134x; 1.2240x over previous
import functools

import jax
import jax.numpy as jnp
from jax import lax
from jax.experimental import pallas as pl
from jax.experimental.pallas import tpu as pltpu
from jax.experimental.pallas import tpu_sc as plsc

N = 10000
E = 160000
D = 128
XP = 16
NLAYERS = 4

CH = 128
NCHUNK = E // CH
NW = 32
CPW = (NCHUNK + NW - 1) // NW
RPS = N // 16
ZR = 25


def _mesh():
    return plsc.VectorSubcoreMesh(core_axis_name="c", subcore_axis_name="s")


def _sc_gather(ph, pc, xp, row2, col2):
    nchunk = row2.shape[0]
    ne = nchunk * CH
    cpw = (nchunk + NW - 1) // NW

    @functools.partial(
        pl.kernel,
        out_type=(
            jax.ShapeDtypeStruct((ne, D), jnp.float32),
            jax.ShapeDtypeStruct((ne, D), jnp.float32),
            jax.ShapeDtypeStruct((ne, XP), jnp.float32),
            jax.ShapeDtypeStruct((ne, XP), jnp.float32),
        ),
        mesh=_mesh(),
        compiler_params=pltpu.CompilerParams(use_tc_tiling_on_sc=False),
        scratch_types=[
            pltpu.VMEM((1, CH), jnp.int32),
            pltpu.VMEM((1, CH), jnp.int32),
            pltpu.VMEM((CH, D), jnp.float32),
            pltpu.VMEM((CH, D), jnp.float32),
            pltpu.VMEM((CH, XP), jnp.float32),
            pltpu.VMEM((CH, XP), jnp.float32),
        ],
    )
    def gk(ph_h, pc_h, xp_h, row_h, col_h,
           phr_h, pcc_h, xr_h, xc_h,
           rowb, colb, bh1, bh2, bx1, bx2):
        c = lax.axis_index("c")
        s = lax.axis_index("s")
        wid = s * 2 + c

        @pl.loop(0, cpw)
        def _(k):
            j = wid + k * NW

            @pl.when(j < nchunk)
            def _():
                base = j * CH
                pltpu.sync_copy(row_h.at[pl.ds(j, 1)], rowb)
                pltpu.sync_copy(col_h.at[pl.ds(j, 1)], colb)
                pltpu.sync_copy(ph_h.at[rowb.at[0]], bh1)
                pltpu.sync_copy(pc_h.at[colb.at[0]], bh2)
                pltpu.sync_copy(xp_h.at[rowb.at[0]], bx1)
                pltpu.sync_copy(xp_h.at[colb.at[0]], bx2)
                pltpu.sync_copy(bh1, phr_h.at[pl.ds(base, CH)])
                pltpu.sync_copy(bh2, pcc_h.at[pl.ds(base, CH)])
                pltpu.sync_copy(bx1, xr_h.at[pl.ds(base, CH)])
                pltpu.sync_copy(bx2, xc_h.at[pl.ds(base, CH)])

    return gk(ph, pc, xp, row2, col2)


def _sc_scatter(m, t16, row2):
    nchunk = row2.shape[0]
    cpw = (nchunk + NW - 1) // NW

    @functools.partial(
        pl.kernel,
        out_type=(
            jax.ShapeDtypeStruct((2, N, D), jnp.float32),
            jax.ShapeDtypeStruct((2, N, XP), jnp.float32),
        ),
        mesh=_mesh(),
        compiler_params=pltpu.CompilerParams(use_tc_tiling_on_sc=False),
        scratch_types=[
            pltpu.VMEM((1, CH), jnp.int32),
            pltpu.VMEM((CH, D), jnp.float32),
            pltpu.VMEM((CH, XP), jnp.float32),
            pltpu.VMEM((ZR, D), jnp.float32),
            pltpu.VMEM((ZR, XP), jnp.float32),
            pltpu.VMEM_SHARED((N, D), jnp.float32),
            pltpu.VMEM_SHARED((N, XP), jnp.float32),
        ],
    )
    def sk(m_h, t_h, row_h, agg_h, tagg_h,
           rowb, mb, tb, zd, zx, agg_sh, tagg_sh):
        c = lax.axis_index("c")
        s = lax.axis_index("s")
        wid = s * 2 + c

        @pl.loop(0, ZR)
        def _(r):
            for g in range(D // 16):
                zd[r, pl.ds(g * 16, 16)] = jnp.zeros((16,), jnp.float32)
            zx[r, pl.ds(0, 16)] = jnp.zeros((16,), jnp.float32)

        @pl.loop(0, RPS // ZR)
        def _(kk):
            off = s * RPS + kk * ZR
            pltpu.sync_copy(zd, agg_sh.at[pl.ds(off, ZR)])
            pltpu.sync_copy(zx, tagg_sh.at[pl.ds(off, ZR)])

        plsc.subcore_barrier()

        @pl.loop(0, cpw)
        def _(k):
            j = wid + k * NW

            @pl.when(j < nchunk)
            def _():
                base = j * CH
                pltpu.sync_copy(row_h.at[pl.ds(j, 1)], rowb)
                pltpu.sync_copy(m_h.at[pl.ds(base, CH)], mb)
                pltpu.sync_copy(t_h.at[pl.ds(base, CH)], tb)
                pltpu.sync_copy(mb, agg_sh.at[rowb.at[0]], add=True)
                pltpu.sync_copy(tb, tagg_sh.at[rowb.at[0]], add=True)

        plsc.subcore_barrier()
        off = s * RPS
        pltpu.sync_copy(agg_sh.at[pl.ds(off, RPS)],
                        agg_h.at[c, pl.ds(off, RPS)])
        pltpu.sync_copy(tagg_sh.at[pl.ds(off, RPS)],
                        tagg_h.at[c, pl.ds(off, RPS)])

    return sk(m, t16, row2)


def _silu(v):
    return v * jax.nn.sigmoid(v)


_BE = 2000
_BN = 2000


def _edge_tc(phr, pcc, xr, xc, ea, eoff, w1c, W1dT, W2T, b2, Wc1T, bc1, wc2):
    ne = phr.shape[0]
    def body(phr_r, pcc_r, xr_r, xc_r, ea_r,
             w1c_r, W1dT_r, W2T_r, b2_r, Wc1T_r, bc1_r, wc2_r,
             m_r, t_r):
        d = xr_r[...] - xc_r[...]
        radial = jnp.sum(d * d, axis=1, keepdims=True)
        z1 = (phr_r[...] + pcc_r[...] + radial * w1c_r[...]
              + jnp.dot(ea_r[...], W1dT_r[...],
                        preferred_element_type=jnp.float32))
        a1 = _silu(z1)
        m = _silu(jnp.dot(a1, W2T_r[...],
                          preferred_element_type=jnp.float32) + b2_r[...])
        cc = _silu(jnp.dot(m, Wc1T_r[...],
                           preferred_element_type=jnp.float32) + bc1_r[...])
        sval = jnp.sum(cc * wc2_r[...], axis=1, keepdims=True)
        t = d * sval
        lane = lax.broadcasted_iota(jnp.int32, (_BE, XP), 1)
        t = jnp.where(lane == 3, 1.0, t)
        m_r[...] = m
        t_r[...] = t

    wspec = pl.BlockSpec((D, D), lambda i: (0, 0))
    vspec = pl.BlockSpec((1, D), lambda i: (0, 0))
    return pl.pallas_call(
        body,
        grid=(ne // _BE,),
        in_specs=[
            pl.BlockSpec((_BE, D), lambda i: (i, 0)),
            pl.BlockSpec((_BE, D), lambda i: (i, 0)),
            pl.BlockSpec((_BE, XP), lambda i: (i, 0)),
            pl.BlockSpec((_BE, XP), lambda i: (i, 0)),
            pl.BlockSpec((_BE, D), lambda i: (i + eoff, 0)),
            vspec, wspec, wspec, vspec, wspec, vspec, vspec,
        ],
        out_specs=[
            pl.BlockSpec((_BE, D), lambda i: (i, 0)),
            pl.BlockSpec((_BE, XP), lambda i: (i, 0)),
        ],
        out_shape=[
            jax.ShapeDtypeStruct((ne, D), jnp.float32),
            jax.ShapeDtypeStruct((ne, XP), jnp.float32),
        ],
    )(phr, pcc, xr, xc, ea, w1c, W1dT, W2T, b2, Wc1T, bc1, wc2)


def _node_tc(h, xp, pa, pt, pa2, pt2,
             n1hT, n1aT, b1n, n2T, b2n, WaT, ba, WbT):
    def body(h_r, xp_r, pa_r, pt_r, pa2_r, pt2_r,
             n1hT_r, n1aT_r, b1n_r, n2T_r, b2n_r, WaT_r, ba_r, WbT_r,
             hn_r, xn_r, ph_r, pc_r):
        agg = pa_r[0] + pa_r[1] + pa2_r[0] + pa2_r[1]
        ts = pt_r[0] + pt_r[1] + pt2_r[0] + pt2_r[1]
        cnt = jnp.maximum(ts[:, 3:4], 1.0)
        lane = lax.broadcasted_iota(jnp.int32, (_BN, XP), 1)
        xn_r[...] = xp_r[...] + jnp.where(lane < 3, ts / cnt, 0.0)
        h = h_r[...]
        z = (jnp.dot(h, n1hT_r[...], preferred_element_type=jnp.float32)
             + jnp.dot(agg, n1aT_r[...], preferred_element_type=jnp.float32)
             + b1n_r[...])
        hn = h + jnp.dot(_silu(z), n2T_r[...],
                         preferred_element_type=jnp.float32) + b2n_r[...]
        hn_r[...] = hn
        ph_r[...] = jnp.dot(hn, WaT_r[...],
                            preferred_element_type=jnp.float32) + ba_r[...]
        pc_r[...] = jnp.dot(hn, WbT_r[...],
                            preferred_element_type=jnp.float32)

    wspec = pl.BlockSpec((D, D), lambda i: (0, 0))
    vspec = pl.BlockSpec((1, D), lambda i: (0, 0))
    return pl.pallas_call(
        body,
        grid=(N // _BN,),
        in_specs=[
            pl.BlockSpec((_BN, D), lambda i: (i, 0)),
            pl.BlockSpec((_BN, XP), lambda i: (i, 0)),
            pl.BlockSpec((2, _BN, D), lambda i: (0, i, 0)),
            pl.BlockSpec((2, _BN, XP), lambda i: (0, i, 0)),
            pl.BlockSpec((2, _BN, D), lambda i: (0, i, 0)),
            pl.BlockSpec((2, _BN, XP), lambda i: (0, i, 0)),
            wspec, wspec, vspec, wspec, vspec, wspec, vspec, wspec,
        ],
        out_specs=[
            pl.BlockSpec((_BN, D), lambda i: (i, 0)),
            pl.BlockSpec((_BN, XP), lambda i: (i, 0)),
            pl.BlockSpec((_BN, D), lambda i: (i, 0)),
            pl.BlockSpec((_BN, D), lambda i: (i, 0)),
        ],
        out_shape=[
            jax.ShapeDtypeStruct((N, D), jnp.float32),
            jax.ShapeDtypeStruct((N, XP), jnp.float32),
            jax.ShapeDtypeStruct((N, D), jnp.float32),
            jax.ShapeDtypeStruct((N, D), jnp.float32),
        ],
    )(h, xp, pa, pt, pa2, pt2, n1hT, n1aT, b1n, n2T, b2n, WaT, ba, WbT)


def _prologue_tc(h, WembT, bemb, WaT, ba, WbT):
    def body(h_r, WembT_r, bemb_r, WaT_r, ba_r, WbT_r, h0_r, ph_r, pc_r):
        h0 = jnp.dot(h_r[...], WembT_r[...],
                     preferred_element_type=jnp.float32) + bemb_r[...]
        h0_r[...] = h0
        ph_r[...] = jnp.dot(h0, WaT_r[...],
                            preferred_element_type=jnp.float32) + ba_r[...]
        pc_r[...] = jnp.dot(h0, WbT_r[...],
                            preferred_element_type=jnp.float32)

    wspec = pl.BlockSpec((D, D), lambda i: (0, 0))
    vspec = pl.BlockSpec((1, D), lambda i: (0, 0))
    bspec = pl.BlockSpec((_BN, D), lambda i: (i, 0))
    return pl.pallas_call(
        body,
        grid=(N // _BN,),
        in_specs=[bspec, wspec, vspec, wspec, vspec, wspec],
        out_specs=[bspec, bspec, bspec],
        out_shape=[
            jax.ShapeDtypeStruct((N, D), jnp.float32),
            jax.ShapeDtypeStruct((N, D), jnp.float32),
            jax.ShapeDtypeStruct((N, D), jnp.float32),
        ],
    )(h, WembT, bemb, WaT, ba, WbT)


def _prep_weights(lp):
    W1 = lp["edge1"]["W"]
    return dict(
        WaT=W1[:, :D].T,
        ba=lp["edge1"]["b"][None],
        WbT=W1[:, D:2 * D].T,
        w1c=W1[:, 2 * D][None],
        W1dT=W1[:, 2 * D + 1:].T,
        W2T=lp["edge2"]["W"].T,
        b2=lp["edge2"]["b"][None],
        Wc1T=lp["coord1"]["W"].T,
        bc1=lp["coord1"]["b"][None],
        wc2=lp["coord2"]["W"],
        n1hT=lp["node1"]["W"][:, :D].T,
        n1aT=lp["node1"]["W"][:, D:].T,
        b1n=lp["node1"]["b"][None],
        n2T=lp["node2"]["W"].T,
        b2n=lp["node2"]["b"][None],
    )


HC = NCHUNK // 2
EH = HC * CH


def kernel(h, x, edges, edge_attr, params):
    row2 = edges[0].reshape(NCHUNK, CH)
    col2 = edges[1].reshape(NCHUNK, CH)
    r2a, r2b = row2[:HC], row2[HC:]
    c2a, c2b = col2[:HC], col2[HC:]
    xp = jnp.pad(x, ((0, 0), (0, XP - 3)))
    ws = [_prep_weights(lp) for lp in params["layers"]]
    h0, ph, pc = _prologue_tc(
        h, params["emb"]["W"].T, params["emb"]["b"][None],
        ws[0]["WaT"], ws[0]["ba"], ws[0]["WbT"])
    hcur, xcur = h0, xp
    ea_full, ea_a, ea_b = edge_attr, None, None
    for l in range(NLAYERS):
        w = ws[l]
        ew = (w["w1c"], w["W1dT"], w["W2T"], w["b2"],
              w["Wc1T"], w["bc1"], w["wc2"])
        ga = _sc_gather(ph, pc, xcur, r2a, c2a)
        gb = _sc_gather(ph, pc, xcur, r2b, c2b)
        if l == 0:
            mA, tA = _edge_tc(*ga, ea_full, 0, *ew)
            mB, tB = _edge_tc(*gb, ea_full, EH // _BE, *ew)
        else:
            mA, tA = _edge_tc(*ga, ea_a, 0, *ew)
            mB, tB = _edge_tc(*gb, ea_b, 0, *ew)
        paA, ptA = _sc_scatter(mA, tA, r2a)
        paB, ptB = _sc_scatter(mB, tB, r2b)
        wn = ws[l + 1] if l + 1 < NLAYERS else ws[0]
        hcur, xcur, ph, pc = _node_tc(
            hcur, xcur, paA, ptA, paB, ptB,
            w["n1hT"], w["n1aT"], w["b1n"], w["n2T"], w["b2n"],
            wn["WaT"], wn["ba"], wn["WbT"])
        ea_a, ea_b = mA, mB
    return hcur
